# Initial kernel scaffold; baseline (speedup 1.0000x reference)
import jax
import jax.numpy as jnp
from jax.experimental import pallas as pl


def kernel(x, edge_index, W1, b1, W2, b2, W3, b3):
    raise NotImplementedError("write your pallas kernel here")


# speedup vs baseline: 65.9641x; 65.9641x over previous
import functools

import jax
import jax.numpy as jnp
from jax import lax
from jax.experimental import pallas as pl
from jax.experimental.pallas import tpu as pltpu
from jax.experimental.pallas import tpu_sc as plsc

NC = 2
NS = 16
NW = NC * NS
L = 16


def _sc_mesh():
  return plsc.VectorSubcoreMesh(core_axis_name="c", subcore_axis_name="s")


_SC_PARAMS = pltpu.CompilerParams(
    use_tc_tiling_on_sc=False, needs_layout_passes=False)


def _make_deg_kernel(r, epw):
  @functools.partial(
      pl.kernel,
      mesh=_sc_mesh(),
      out_type=jax.ShapeDtypeStruct((NW, 1, r), jnp.float32),
      compiler_params=_SC_PARAMS,
      scratch_types=[
          pltpu.VMEM((epw,), jnp.int32),
          pltpu.VMEM((r,), jnp.float32),
      ],
  )
  def deg_kernel(dst_h, zeros_h, out_h, dst_v, acc_v):
    cid = lax.axis_index("c")
    sid = lax.axis_index("s")
    wid = cid * NS + sid
    pltpu.sync_copy(dst_h.at[pl.ds(wid * epw, epw)], dst_v)
    pltpu.sync_copy(zeros_h, acc_v)
    ones16 = jnp.ones((L,), jnp.float32)

    def step(i, carry):
      d16 = dst_v[pl.ds(i * L, L)]
      plsc.addupdate_scatter(acc_v, [d16], ones16)
      return carry

    lax.fori_loop(0, epw // L, step, 0)
    pltpu.sync_copy(acc_v, out_h.at[wid, 0])

  return deg_kernel


def _make_scatter_kernel(n, r, epw, f):
  @functools.partial(
      pl.kernel,
      mesh=_sc_mesh(),
      out_type=jax.ShapeDtypeStruct((NW, f, r), jnp.float32),
      compiler_params=_SC_PARAMS,
      scratch_types=[
          pltpu.VMEM((epw,), jnp.int32),
          pltpu.VMEM((epw,), jnp.int32),
          pltpu.VMEM((n,), jnp.float32),
          pltpu.VMEM((r,), jnp.float32),
      ],
  )
  def scatter_kernel(y_h, src_h, dst_h, zeros_h, out_h,
                     src_v, dst_v, yc_v, acc_v):
    cid = lax.axis_index("c")
    sid = lax.axis_index("s")
    wid = cid * NS + sid
    pltpu.sync_copy(src_h.at[pl.ds(wid * epw, epw)], src_v)
    pltpu.sync_copy(dst_h.at[pl.ds(wid * epw, epw)], dst_v)

    for c in range(f):
      pltpu.sync_copy(y_h.at[c], yc_v)
      pltpu.sync_copy(zeros_h, acc_v)

      def step(i, carry):
        s16 = src_v[pl.ds(i * L, L)]
        d16 = dst_v[pl.ds(i * L, L)]
        vals = plsc.load_gather(yc_v, [s16])
        plsc.addupdate_scatter(acc_v, [d16], vals)
        return carry

      lax.fori_loop(0, epw // L, step, 0)
      pltpu.sync_copy(acc_v, out_h.at[wid, c])

  return scatter_kernel


_BLK = 1024


def _ka_body(x_ref, w1t_ref, degp_ref, y1_ref, dinv_ref):
  deg = jnp.sum(degp_ref[...], axis=0) + 1.0
  dinv = lax.rsqrt(deg)
  xwt = lax.dot_general(
      w1t_ref[...], x_ref[...], (((1,), (1,)), ((), ())),
      preferred_element_type=jnp.float32,
      precision=lax.Precision.HIGHEST)
  y1_ref[...] = xwt * dinv
  dinv_ref[...] = dinv


def _ka_call(x, w1t, degp, n, d):
  grid = (-(-n // _BLK),)
  return pl.pallas_call(
      _ka_body,
      grid=grid,
      in_specs=[
          pl.BlockSpec((_BLK, d), lambda i: (i, 0)),
          pl.BlockSpec((4, d), lambda i: (0, 0)),
          pl.BlockSpec((NW, 1, _BLK), lambda i: (0, 0, i)),
      ],
      out_specs=[
          pl.BlockSpec((4, _BLK), lambda i: (0, i)),
          pl.BlockSpec((1, _BLK), lambda i: (0, i)),
      ],
      out_shape=[
          jax.ShapeDtypeStruct((4, n), jnp.float32),
          jax.ShapeDtypeStruct((1, n), jnp.float32),
      ],
  )(x, w1t, degp)


def _glue_body(f_in, f_out, accp_ref, y_ref, dinv_ref, b_ref, wt_ref,
               out_ref):
  dinv = dinv_ref[...]
  acc = jnp.sum(accp_ref[...], axis=0)
  h = jnp.maximum(dinv * (acc + y_ref[...]) + b_ref[...], 0.0)
  yn = wt_ref[:, 0:1] * h[0:1, :]
  for c in range(1, f_in):
    yn = yn + wt_ref[:, c:c + 1] * h[c:c + 1, :]
  out_ref[...] = yn * dinv


def _glue_call(accp, y, dinv, b, wt, n, f_in, f_out):
  grid = (-(-n // _BLK),)
  body = functools.partial(_glue_body, f_in, f_out)
  return pl.pallas_call(
      body,
      grid=grid,
      in_specs=[
          pl.BlockSpec((NW, f_in, _BLK), lambda i: (0, 0, i)),
          pl.BlockSpec((f_in, _BLK), lambda i: (0, i)),
          pl.BlockSpec((1, _BLK), lambda i: (0, i)),
          pl.BlockSpec((f_in, 1), lambda i: (0, 0)),
          pl.BlockSpec((f_out, f_in), lambda i: (0, 0)),
      ],
      out_specs=pl.BlockSpec((f_out, _BLK), lambda i: (0, i)),
      out_shape=jax.ShapeDtypeStruct((f_out, n), jnp.float32),
  )(accp, y, dinv, b, wt)


def _kd_body(accp_ref, y3_ref, dinv_ref, b3_ref, out_ref):
  acc = jnp.sum(accp_ref[...], axis=0)
  out_ref[...] = jnp.tanh(
      dinv_ref[...] * (acc + y3_ref[...]) + b3_ref[...])


def _kd_call(accp, y3, dinv, b3, n):
  grid = (-(-n // _BLK),)
  return pl.pallas_call(
      _kd_body,
      grid=grid,
      in_specs=[
          pl.BlockSpec((NW, 1, _BLK), lambda i: (0, 0, i)),
          pl.BlockSpec((1, _BLK), lambda i: (0, i)),
          pl.BlockSpec((1, _BLK), lambda i: (0, i)),
          pl.BlockSpec((1, 1), lambda i: (0, 0)),
      ],
      out_specs=pl.BlockSpec((1, _BLK), lambda i: (0, i)),
      out_shape=jax.ShapeDtypeStruct((1, n), jnp.float32),
  )(accp, y3, dinv, b3)


def kernel(x, edge_index, W1, b1, W2, b2, W3, b3):
  n, d = x.shape
  e = edge_index.shape[1]

  ndum = 512
  r = -(-(n + ndum) // 8) * 8
  epw = -(-e // (NW * L * 8)) * (L * 8)
  pad = epw * NW - e
  src = edge_index[0]
  dst = edge_index[1]
  srcp = jnp.concatenate([src, jnp.zeros((pad,), jnp.int32)])
  dstp = jnp.concatenate(
      [dst, n + (jnp.arange(pad, dtype=jnp.int32) % ndum)])

  zeros_r = jnp.zeros((r,), jnp.float32)
  w1t = W1.T
  w2t = W2.T
  w3t = W3.T
  b1r = b1.reshape(4, 1)
  b2r = b2.reshape(2, 1)
  b3r = b3.reshape(1, 1)

  degp = _make_deg_kernel(r, epw)(dstp, zeros_r)
  y1, dinv = _ka_call(x, w1t, degp, n, d)
  acc1p = _make_scatter_kernel(n, r, epw, 4)(y1, srcp, dstp, zeros_r)
  y2 = _glue_call(acc1p, y1, dinv, b1r, w2t, n, 4, 2)
  acc2p = _make_scatter_kernel(n, r, epw, 2)(y2, srcp, dstp, zeros_r)
  y3 = _glue_call(acc2p, y2, dinv, b2r, w3t, n, 2, 1)
  acc3p = _make_scatter_kernel(n, r, epw, 1)(y3, srcp, dstp, zeros_r)
  out = _kd_call(acc3p, y3, dinv, b3r, n)
  return out.reshape(n, 1)


# speedup vs baseline: 67.4029x; 1.0218x over previous
import functools

import jax
import jax.numpy as jnp
from jax import lax
from jax.experimental import pallas as pl
from jax.experimental.pallas import tpu as pltpu
from jax.experimental.pallas import tpu_sc as plsc

NC = 2
NS = 16
NW = NC * NS
L = 16


def _sc_mesh():
  return plsc.VectorSubcoreMesh(core_axis_name="c", subcore_axis_name="s")


_SC_PARAMS = pltpu.CompilerParams(
    use_tc_tiling_on_sc=False, needs_layout_passes=False)


def _make_deg_kernel(r, epw):
  @functools.partial(
      pl.kernel,
      mesh=_sc_mesh(),
      out_type=jax.ShapeDtypeStruct((NW, 1, r), jnp.float32),
      compiler_params=_SC_PARAMS,
      scratch_types=[
          pltpu.VMEM((epw,), jnp.int32),
          pltpu.VMEM((r,), jnp.float32),
      ],
  )
  def deg_kernel(dst_h, zeros_h, out_h, dst_v, acc_v):
    cid = lax.axis_index("c")
    sid = lax.axis_index("s")
    wid = cid * NS + sid
    pltpu.sync_copy(dst_h.at[pl.ds(wid * epw, epw)], dst_v)
    pltpu.sync_copy(zeros_h, acc_v)
    ones16 = jnp.ones((L,), jnp.float32)
    u = 8

    def step(i, carry):
      base = i * (L * u)
      for k in range(u):
        d16 = dst_v[pl.ds(base + k * L, L)]
        plsc.addupdate_scatter(acc_v, [d16], ones16)
      return carry

    lax.fori_loop(0, epw // (L * u), step, 0)
    pltpu.sync_copy(acc_v, out_h.at[wid, 0])

  return deg_kernel


def _make_scatter_kernel(n, r, epw, f):
  @functools.partial(
      pl.kernel,
      mesh=_sc_mesh(),
      out_type=jax.ShapeDtypeStruct((NW, f, r), jnp.float32),
      compiler_params=_SC_PARAMS,
      scratch_types=[
          pltpu.VMEM((epw,), jnp.int32),
          pltpu.VMEM((epw,), jnp.int32),
          pltpu.VMEM((n,), jnp.float32),
          pltpu.VMEM((r,), jnp.float32),
      ],
  )
  def scatter_kernel(y_h, src_h, dst_h, zeros_h, out_h,
                     src_v, dst_v, yc_v, acc_v):
    cid = lax.axis_index("c")
    sid = lax.axis_index("s")
    wid = cid * NS + sid
    pltpu.sync_copy(src_h.at[pl.ds(wid * epw, epw)], src_v)
    pltpu.sync_copy(dst_h.at[pl.ds(wid * epw, epw)], dst_v)

    for c in range(f):
      pltpu.sync_copy(y_h.at[c], yc_v)
      pltpu.sync_copy(zeros_h, acc_v)

      u = 8

      def step(i, carry):
        base = i * (L * u)
        for k in range(u):
          s16 = src_v[pl.ds(base + k * L, L)]
          d16 = dst_v[pl.ds(base + k * L, L)]
          vals = plsc.load_gather(yc_v, [s16])
          plsc.addupdate_scatter(acc_v, [d16], vals)
        return carry

      lax.fori_loop(0, epw // (L * u), step, 0)
      pltpu.sync_copy(acc_v, out_h.at[wid, c])

  return scatter_kernel


_BLK = 2048


def _ka_body(x_ref, w1t_ref, degp_ref, y1_ref, dinv_ref):
  deg = jnp.sum(degp_ref[...], axis=0) + 1.0
  dinv = lax.rsqrt(deg)
  xwt = lax.dot_general(
      w1t_ref[...], x_ref[...], (((1,), (1,)), ((), ())),
      preferred_element_type=jnp.float32,
      precision=lax.Precision.HIGHEST)
  y1_ref[...] = xwt * dinv
  dinv_ref[...] = dinv


def _ka_call(x, w1t, degp, n, d):
  grid = (-(-n // _BLK),)
  return pl.pallas_call(
      _ka_body,
      grid=grid,
      in_specs=[
          pl.BlockSpec((_BLK, d), lambda i: (i, 0)),
          pl.BlockSpec((4, d), lambda i: (0, 0)),
          pl.BlockSpec((NW, 1, _BLK), lambda i: (0, 0, i)),
      ],
      out_specs=[
          pl.BlockSpec((4, _BLK), lambda i: (0, i)),
          pl.BlockSpec((1, _BLK), lambda i: (0, i)),
      ],
      out_shape=[
          jax.ShapeDtypeStruct((4, n), jnp.float32),
          jax.ShapeDtypeStruct((1, n), jnp.float32),
      ],
  )(x, w1t, degp)


def _glue_body(f_in, f_out, accp_ref, y_ref, dinv_ref, b_ref, wt_ref,
               out_ref):
  dinv = dinv_ref[...]
  acc = jnp.sum(accp_ref[...], axis=0)
  h = jnp.maximum(dinv * (acc + y_ref[...]) + b_ref[...], 0.0)
  yn = wt_ref[:, 0:1] * h[0:1, :]
  for c in range(1, f_in):
    yn = yn + wt_ref[:, c:c + 1] * h[c:c + 1, :]
  out_ref[...] = yn * dinv


def _glue_call(accp, y, dinv, b, wt, n, f_in, f_out):
  grid = (-(-n // _BLK),)
  body = functools.partial(_glue_body, f_in, f_out)
  return pl.pallas_call(
      body,
      grid=grid,
      in_specs=[
          pl.BlockSpec((NW, f_in, _BLK), lambda i: (0, 0, i)),
          pl.BlockSpec((f_in, _BLK), lambda i: (0, i)),
          pl.BlockSpec((1, _BLK), lambda i: (0, i)),
          pl.BlockSpec((f_in, 1), lambda i: (0, 0)),
          pl.BlockSpec((f_out, f_in), lambda i: (0, 0)),
      ],
      out_specs=pl.BlockSpec((f_out, _BLK), lambda i: (0, i)),
      out_shape=jax.ShapeDtypeStruct((f_out, n), jnp.float32),
  )(accp, y, dinv, b, wt)


def _kd_body(accp_ref, y3_ref, dinv_ref, b3_ref, out_ref):
  acc = jnp.sum(accp_ref[...], axis=0)
  res = jnp.tanh(
      dinv_ref[...] * (acc + y3_ref[...]) + b3_ref[...])
  out_ref[...] = jnp.reshape(res, (_BLK, 1))


def _kd_call(accp, y3, dinv, b3, n):
  grid = (-(-n // _BLK),)
  return pl.pallas_call(
      _kd_body,
      grid=grid,
      in_specs=[
          pl.BlockSpec((NW, 1, _BLK), lambda i: (0, 0, i)),
          pl.BlockSpec((1, _BLK), lambda i: (0, i)),
          pl.BlockSpec((1, _BLK), lambda i: (0, i)),
          pl.BlockSpec((1, 1), lambda i: (0, 0)),
      ],
      out_specs=pl.BlockSpec((_BLK, 1), lambda i: (i, 0)),
      out_shape=jax.ShapeDtypeStruct((n, 1), jnp.float32),
  )(accp, y3, dinv, b3)


def kernel(x, edge_index, W1, b1, W2, b2, W3, b3):
  n, d = x.shape
  e = edge_index.shape[1]

  ndum = 512
  r = -(-(n + ndum) // 8) * 8
  epw = -(-e // (NW * L * 8)) * (L * 8)
  pad = epw * NW - e
  src = edge_index[0]
  dst = edge_index[1]
  srcp = jnp.concatenate([src, jnp.zeros((pad,), jnp.int32)])
  dstp = jnp.concatenate(
      [dst, n + (jnp.arange(pad, dtype=jnp.int32) % ndum)])

  zeros_r = jnp.zeros((r,), jnp.float32)
  w1t = W1.T
  w2t = W2.T
  w3t = W3.T
  b1r = b1.reshape(4, 1)
  b2r = b2.reshape(2, 1)
  b3r = b3.reshape(1, 1)

  degp = _make_deg_kernel(r, epw)(dstp, zeros_r)
  y1, dinv = _ka_call(x, w1t, degp, n, d)
  acc1p = _make_scatter_kernel(n, r, epw, 4)(y1, srcp, dstp, zeros_r)
  y2 = _glue_call(acc1p, y1, dinv, b1r, w2t, n, 4, 2)
  acc2p = _make_scatter_kernel(n, r, epw, 2)(y2, srcp, dstp, zeros_r)
  y3 = _glue_call(acc2p, y2, dinv, b2r, w3t, n, 2, 1)
  acc3p = _make_scatter_kernel(n, r, epw, 1)(y3, srcp, dstp, zeros_r)
  return _kd_call(acc3p, y3, dinv, b3r, n)


# speedup vs baseline: 120.4362x; 1.7868x over previous
import functools

import jax
import jax.numpy as jnp
from jax import lax
from jax.experimental import pallas as pl
from jax.experimental.pallas import tpu as pltpu
from jax.experimental.pallas import tpu_sc as plsc

NC = 2
NS = 16
NW = NC * NS
L = 16


def _sc_mesh():
  return plsc.VectorSubcoreMesh(core_axis_name="c", subcore_axis_name="s")


_SC_PARAMS = pltpu.CompilerParams(
    use_tc_tiling_on_sc=False, needs_layout_passes=False)


def _reduce_and_emit(acc_v, stack_sh, stack_v, sum_v, out_slice, sid, stripe):
  pltpu.sync_copy(acc_v, stack_sh.at[sid])
  plsc.subcore_barrier()
  pltpu.sync_copy(stack_sh.at[:, pl.ds(sid * stripe, stripe)], stack_v)

  def rstep(j, carry):
    tot = stack_v[0, pl.ds(j * L, L)]
    for s in range(1, NS):
      tot = tot + stack_v[s, pl.ds(j * L, L)]
    sum_v[pl.ds(j * L, L)] = tot
    return carry

  lax.fori_loop(0, stripe // L, rstep, 0)
  pltpu.sync_copy(sum_v, out_slice)
  plsc.subcore_barrier()


def _make_deg_kernel(r, epw):
  stripe = r // NS

  @functools.partial(
      pl.kernel,
      mesh=_sc_mesh(),
      out_type=jax.ShapeDtypeStruct((NC, 1, r), jnp.float32),
      compiler_params=_SC_PARAMS,
      scratch_types=[
          pltpu.VMEM((epw,), jnp.int32),
          pltpu.VMEM((r,), jnp.float32),
          pltpu.VMEM((NS, stripe), jnp.float32),
          pltpu.VMEM((stripe,), jnp.float32),
          pltpu.VMEM_SHARED((NS, r), jnp.float32),
      ],
  )
  def deg_kernel(ei_h, zeros_h, out_h, dst_v, acc_v, stack_v, sum_v,
                 stack_sh):
    cid = lax.axis_index("c")
    sid = lax.axis_index("s")
    wid = cid * NS + sid
    pltpu.sync_copy(ei_h.at[1, pl.ds(wid * epw, epw)], dst_v)
    pltpu.sync_copy(zeros_h, acc_v)
    ones16 = jnp.ones((L,), jnp.float32)
    u = 8
    n_main = epw // (L * u)

    def step(i, carry):
      base = i * (L * u)
      ds = [dst_v[pl.ds(base + k * L, L)] for k in range(u)]
      for k in range(u):
        plsc.addupdate_scatter(acc_v, [ds[k]], ones16)
      return carry

    lax.fori_loop(0, n_main, step, 0)
    for t in range(n_main * u, epw // L):
      d16 = dst_v[pl.ds(t * L, L)]
      plsc.addupdate_scatter(acc_v, [d16], ones16)
    _reduce_and_emit(acc_v, stack_sh, stack_v, sum_v,
                     out_h.at[cid, 0, pl.ds(sid * stripe, stripe)],
                     sid, stripe)

  return deg_kernel


def _make_scatter_kernel(n, r, epw, f):
  stripe = r // NS

  g = 2 if f % 2 == 0 else 1
  np_ = f // g

  @functools.partial(
      pl.kernel,
      mesh=_sc_mesh(),
      out_type=jax.ShapeDtypeStruct((NC, f, r), jnp.float32),
      compiler_params=_SC_PARAMS,
      scratch_types=[
          pltpu.VMEM((epw,), jnp.int32),
          pltpu.VMEM((epw,), jnp.int32),
          pltpu.VMEM((g, n), jnp.float32),
          pltpu.VMEM((g, n), jnp.float32),
          pltpu.VMEM((g, r), jnp.float32),
          pltpu.VMEM((NS, g, stripe), jnp.float32),
          pltpu.VMEM((g, stripe), jnp.float32),
          pltpu.VMEM_SHARED((NS, g, r), jnp.float32),
          pltpu.SemaphoreType.DMA,
          pltpu.SemaphoreType.DMA,
          pltpu.SemaphoreType.DMA,
          pltpu.SemaphoreType.DMA,
      ],
  )
  def scatter_kernel(y_h, ei_h, zeros_h, out_h,
                     src_v, dst_v, yc0_v, yc1_v, acc_v, stack_v, sum_v,
                     stack_sh, sem_s, sem_d, sem_y, sem_z):
    cid = lax.axis_index("c")
    sid = lax.axis_index("s")
    wid = cid * NS + sid
    ycs = [yc0_v, yc1_v]
    cp = [
        pltpu.async_copy(ei_h.at[0, pl.ds(wid * epw, epw)], src_v, sem_s),
        pltpu.async_copy(ei_h.at[1, pl.ds(wid * epw, epw)], dst_v, sem_d),
        pltpu.async_copy(y_h.at[pl.ds(0, g)], ycs[0], sem_y),
        pltpu.async_copy(zeros_h, acc_v, sem_z),
    ]
    for p in cp:
      p.wait()
    cvecs = [jnp.full((L,), cc, jnp.int32) for cc in range(g)]

    for c in range(np_):
      yc_v = ycs[c % 2]
      u = 8
      n_main = epw // (L * u)

      def step(i, carry):
        base = i * (L * u)
        ss = [src_v[pl.ds(base + k * L, L)] for k in range(u)]
        ds = [dst_v[pl.ds(base + k * L, L)] for k in range(u)]
        vs = [[plsc.load_gather(yc_v, [cvecs[cc], ss[k]]) for cc in range(g)]
              for k in range(u)]
        for k in range(u):
          for cc in range(g):
            plsc.addupdate_scatter(acc_v, [cvecs[cc], ds[k]], vs[k][cc])
        return carry

      lax.fori_loop(0, n_main, step, 0)
      for t in range(n_main * u, epw // L):
        s16 = src_v[pl.ds(t * L, L)]
        d16 = dst_v[pl.ds(t * L, L)]
        for cc in range(g):
          vals = plsc.load_gather(yc_v, [cvecs[cc], s16])
          plsc.addupdate_scatter(acc_v, [cvecs[cc], d16], vals)

      pltpu.sync_copy(acc_v, stack_sh.at[sid])
      pend = []
      if c + 1 < np_:
        pend.append(pltpu.async_copy(y_h.at[pl.ds((c + 1) * g, g)],
                                     ycs[(c + 1) % 2], sem_y))
        pend.append(pltpu.async_copy(zeros_h, acc_v, sem_z))
      plsc.subcore_barrier()
      pltpu.sync_copy(
          stack_sh.at[:, :, pl.ds(sid * stripe, stripe)], stack_v)

      def rstep(j, carry):
        for cc in range(g):
          tot = stack_v[0, cc, pl.ds(j * L, L)]
          for s in range(1, NS):
            tot = tot + stack_v[s, cc, pl.ds(j * L, L)]
          sum_v[cc, pl.ds(j * L, L)] = tot
        return carry

      lax.fori_loop(0, stripe // L, rstep, 0)
      pltpu.sync_copy(
          sum_v, out_h.at[cid, pl.ds(c * g, g), pl.ds(sid * stripe, stripe)])
      for p in pend:
        p.wait()
      plsc.subcore_barrier()

  return scatter_kernel


_BLK = 2048


def _ka1_body(x_ref, w1_ref, xwt_ref):
  xw = jnp.dot(x_ref[...], w1_ref[...],
               preferred_element_type=jnp.float32)
  xwt_ref[...] = xw.T


def _ka1_call(x, w1, n):
  return pl.pallas_call(
      _ka1_body,
      out_shape=jax.ShapeDtypeStruct((4, n), jnp.float32),
  )(x, w1)


def _ka2_body(n, xwt_ref, degp_ref, y1_ref, dinv_ref):
  deg = jnp.sum(degp_ref[...], axis=0)[:, :n] + 1.0
  dinv = lax.rsqrt(deg)
  y1_ref[...] = xwt_ref[...] * dinv
  dinv_ref[...] = dinv


def _ka2_call(xwt, degp, n):
  return pl.pallas_call(
      functools.partial(_ka2_body, n),
      out_shape=[
          jax.ShapeDtypeStruct((4, n), jnp.float32),
          jax.ShapeDtypeStruct((1, n), jnp.float32),
      ],
  )(xwt, degp)


def _glue_body(n, f_in, f_out, accp_ref, y_ref, dinv_ref, b_ref, wt_ref,
               out_ref):
  dinv = dinv_ref[...]
  acc = jnp.sum(accp_ref[...], axis=0)[:, :n]
  h = jnp.maximum(dinv * (acc + y_ref[...]) + b_ref[...], 0.0)
  yn = wt_ref[:, 0:1] * h[0:1, :]
  for c in range(1, f_in):
    yn = yn + wt_ref[:, c:c + 1] * h[c:c + 1, :]
  out_ref[...] = yn * dinv


def _glue_call(accp, y, dinv, b, wt, n, f_in, f_out):
  body = functools.partial(_glue_body, n, f_in, f_out)
  return pl.pallas_call(
      body,
      out_shape=jax.ShapeDtypeStruct((f_out, n), jnp.float32),
  )(accp, y, dinv, b, wt)


def _kd_body(n, accp_ref, y3_ref, dinv_ref, b3_ref, out_ref):
  acc = jnp.sum(accp_ref[...], axis=0)[:, :n]
  res = jnp.tanh(
      dinv_ref[...] * (acc + y3_ref[...]) + b3_ref[...])
  out_ref[...] = jnp.reshape(res, (n, 1))


def _kd_call(accp, y3, dinv, b3, n):
  return pl.pallas_call(
      functools.partial(_kd_body, n),
      out_shape=jax.ShapeDtypeStruct((n, 1), jnp.float32),
  )(accp, y3, dinv, b3)


def kernel(x, edge_index, W1, b1, W2, b2, W3, b3):
  n, d = x.shape
  e = edge_index.shape[1]

  ndum = 512
  r = NS * (-(-(n + ndum) // (NS * L)) * L)
  if e % (NW * L) == 0:
    ei = edge_index
  else:
    pad = -(-e // (NW * L)) * (NW * L) - e
    src = jnp.concatenate([edge_index[0], jnp.zeros((pad,), jnp.int32)])
    dst = jnp.concatenate(
        [edge_index[1], n + (jnp.arange(pad, dtype=jnp.int32) % ndum)])
    ei = jnp.stack([src, dst])
  epw = ei.shape[1] // NW

  zeros_r = jnp.zeros((r,), jnp.float32)
  zeros_2r = jnp.zeros((2, r), jnp.float32)
  zeros_1r = jnp.zeros((1, r), jnp.float32)
  w2t = W2.T
  w3t = W3.T
  b1r = b1.reshape(4, 1)
  b2r = b2.reshape(2, 1)
  b3r = b3.reshape(1, 1)

  degp = _make_deg_kernel(r, epw)(ei, zeros_r)
  xwt = _ka1_call(x, W1, n)
  y1, dinv = _ka2_call(xwt, degp, n)
  acc1p = _make_scatter_kernel(n, r, epw, 4)(y1, ei, zeros_2r)
  y2 = _glue_call(acc1p, y1, dinv, b1r, w2t, n, 4, 2)
  acc2p = _make_scatter_kernel(n, r, epw, 2)(y2, ei, zeros_2r)
  y3 = _glue_call(acc2p, y2, dinv, b2r, w3t, n, 2, 1)
  acc3p = _make_scatter_kernel(n, r, epw, 1)(y3, ei, zeros_1r)
  return _kd_call(acc3p, y3, dinv, b3r, n)
